# baseline (speedup 1.0000x reference)
import functools
import jax
import jax.numpy as jnp
import numpy as np
from jax.experimental import pallas as pl
from jax.experimental.pallas import tpu as pltpu
from jax.experimental.pallas import tpu_sc as plsc

K = 16
CUTOFF = 5.0
SCALE = 0.1
LN_EPS = 1e-5


def _gelu(v):
    return 0.5 * v * (1.0 + jax.lax.erf(v * np.float32(0.7071067811865476)))


def _ln(v, g, b):
    mu = jnp.mean(v, axis=-1, keepdims=True)
    var = jnp.mean((v - mu) * (v - mu), axis=-1, keepdims=True)
    return (v - mu) * jax.lax.rsqrt(var + LN_EPS) * g + b


def _knn_embed_kernel(x_blk_ref, x_full_ref, win_ref, bin_ref,
                      h_ref, idx_ref, d_ref, *, blk, n):
    b = pl.program_id(0)
    i = pl.program_id(1)
    xb = x_blk_ref[0]
    xf = x_full_ref[0]
    pos_b = xb[:, :3]
    pos_f = xf[:, :3]
    sq_b = jnp.sum(pos_b * pos_b, axis=1)
    sq_f = jnp.sum(pos_f * pos_f, axis=1)
    cross = jax.lax.dot_general(pos_b, pos_f, (((1,), (1,)), ((), ())),
                                preferred_element_type=jnp.float32)
    d2 = sq_b[:, None] + sq_f[None, :] - 2.0 * cross
    cols = jax.lax.broadcasted_iota(jnp.int32, (blk, n), 1)
    rows = jax.lax.broadcasted_iota(jnp.int32, (blk, n), 0) + i * blk
    d2 = jnp.where(cols == rows, d2 + 1e9, d2)
    idxs = []
    vals = []
    for _ in range(K):
        m = jnp.min(d2, axis=1)
        a = jnp.argmin(d2, axis=1).astype(jnp.int32)
        idxs.append(a[:, None])
        vals.append(m[:, None])
        d2 = jnp.where(cols == a[:, None], jnp.float32(3e38), d2)
    idx_ref[0] = jnp.concatenate(idxs, axis=1) + b * n
    d2k = jnp.concatenate(vals, axis=1)
    d_ref[0] = jnp.sqrt(jnp.maximum(d2k, 1e-12))
    h_ref[0] = jnp.dot(xb, win_ref[...], preferred_element_type=jnp.float32) + bin_ref[...]


def _layer_kernel(h_blk_ref, h_full_ref, hj_ref, d_ref, centers_ref,
                  we1a_ref, we1b_ref, we1r_ref, be1_ref,
                  we2_ref, be2_ref, wu_ref, bu_ref, lnlg_ref, lnlb_ref,
                  wp_ref, bp_ref, wm_ref, bm_ref, wi_ref, bi_ref,
                  lngg_ref, lngb_ref, wf1_ref, wf2_ref, bf_ref,
                  wc1_ref, bc1_ref, wc2_ref, bc2_ref,
                  wg_ref, bgt_ref, n1g_ref, n1b_ref, n2g_ref, n2b_ref,
                  out_ref, *, blk, gamma, d_model):
    D = d_model
    h = h_blk_ref[0]
    hj = hj_ref[0]
    d = d_ref[0]

    r_centers = centers_ref[...][0]
    rbf = jnp.exp(-gamma * (d[:, :, None] - r_centers) ** 2)
    nr = r_centers.shape[-1]
    hj2 = hj.reshape(blk * K, D)
    t = jax.lax.dot_general(hj2, we1b_ref[...], (((1,), (0,)), ((), ())),
                            preferred_element_type=jnp.float32)
    t = t + jax.lax.dot_general(rbf.reshape(blk * K, nr), we1r_ref[...],
                                (((1,), (0,)), ((), ())),
                                preferred_element_type=jnp.float32)
    hi_part = jnp.dot(h, we1a_ref[...], preferred_element_type=jnp.float32)
    pre = t.reshape(blk, K, D) + hi_part[:, None, :] + be1_ref[...][None]
    m1 = pre * jax.nn.sigmoid(pre)
    m = jnp.dot(m1.reshape(blk * K, D), we2_ref[...],
                preferred_element_type=jnp.float32) + be2_ref[...]
    mask = (d <= CUTOFF).astype(jnp.float32)
    m = m.reshape(blk, K, D) * mask[:, :, None]
    agg = jnp.sum(m, axis=1)
    local = _ln(h + jnp.dot(agg, wu_ref[...], preferred_element_type=jnp.float32)
                + bu_ref[...], lnlg_ref[...], lnlb_ref[...])

    hf = h_full_ref[0]
    mf = jnp.mean(hf, axis=0, keepdims=True)
    pmv = jnp.dot(h, wp_ref[...], preferred_element_type=jnp.float32) + bp_ref[...]
    mmv = jnp.dot(mf, wm_ref[...], preferred_element_type=jnp.float32) + bm_ref[...]
    mmv_t = jnp.concatenate([mmv] * 8, axis=1)
    rows8 = jax.lax.broadcasted_iota(jnp.int32, (8, 64), 0)
    cols64 = jax.lax.broadcasted_iota(jnp.int32, (8, 64), 1)
    msel = (cols64 // 8 == rows8).astype(jnp.float32)
    M = mmv_t * msel
    Q = jnp.dot(M, wi_ref[...], preferred_element_type=jnp.float32)
    glob_delta = jnp.dot(pmv, Q, preferred_element_type=jnp.float32) + bi_ref[...]
    glob = _ln(h + SCALE * glob_delta, lngg_ref[...], lngb_ref[...])

    fpre = (jnp.dot(local, wf1_ref[...], preferred_element_type=jnp.float32)
            + jnp.dot(glob, wf2_ref[...], preferred_element_type=jnp.float32)
            + bf_ref[...])
    fused = _gelu(fpre)

    v = _ln(fused, n1g_ref[...], n1b_ref[...])
    v = _gelu(jnp.dot(v, wc1_ref[...], preferred_element_type=jnp.float32)
              + bc1_ref[...])
    v = jnp.dot(v, wc2_ref[...], preferred_element_type=jnp.float32) + bc2_ref[...]
    v = v + fused

    ng = D // 8
    rid = jax.lax.broadcasted_iota(jnp.int32, (D, D), 0)
    cid = jax.lax.broadcasted_iota(jnp.int32, (D, D), 1)
    same_grp = (rid // 8 == cid // 8)
    geo = jnp.zeros((blk, D), jnp.float32)
    wg = wg_ref[...]
    for i in range(8):
        ei = (same_grp & (rid % 8 == i)).astype(jnp.float32)
        vi = jnp.dot(v, ei, preferred_element_type=jnp.float32)
        wgi = wg[8 * i:8 * (i + 1), :]
        tiled = jnp.concatenate([jnp.concatenate([wgi] * ng, axis=1)] * ng, axis=0)
        di = tiled * same_grp.astype(jnp.float32)
        ti = jnp.dot(v, di, preferred_element_type=jnp.float32)
        geo = geo + vi * ti
    geo = geo + bgt_ref[...]
    out_ref[0] = _ln(v + SCALE * geo, n2g_ref[...], n2b_ref[...])


def _make_sc_gather(total_rows, d_model):
    info = plsc.get_sparse_core_info()
    nc, ns = info.num_cores, info.num_subcores
    nw = nc * ns
    irows = total_rows // 128
    rpw = irows // nw
    mesh = plsc.VectorSubcoreMesh(core_axis_name="c", subcore_axis_name="s")

    @functools.partial(
        pl.kernel, mesh=mesh,
        out_type=jax.ShapeDtypeStruct((total_rows, d_model), jnp.float32),
        scratch_types=[
            pltpu.VMEM((rpw, 128), jnp.int32),
            pltpu.VMEM((128, d_model), jnp.float32),
            pltpu.SemaphoreType.DMA,
        ],
    )
    def gather(table_hbm, idx_hbm, out_hbm, idx_v, rows_v, sem):
        c = jax.lax.axis_index("c")
        s = jax.lax.axis_index("s")
        wid = s * nc + c
        pltpu.sync_copy(idx_hbm.at[pl.ds(wid * rpw, rpw)], idx_v)

        def body(j, carry):
            pltpu.async_copy(table_hbm.at[idx_v.at[j]], rows_v, sem).wait()
            pltpu.sync_copy(
                rows_v, out_hbm.at[pl.ds((wid * rpw + j) * 128, 128)])
            return carry

        jax.lax.fori_loop(0, rpw, body, 0)

    return gather


def _out_kernel(x_blk_ref, h_blk_ref, wout_ref, bout_ref, o_ref):
    o_ref[0] = (x_blk_ref[0]
                + jnp.dot(h_blk_ref[0], wout_ref[...],
                          preferred_element_type=jnp.float32)
                + bout_ref[...])


def kernel(x, params):
    p = params
    B, N, _ = x.shape
    D = p['Win'].shape[1]
    L = p['We1'].shape[0]
    R = p['We1'].shape[1] - 2 * D
    blk = min(256, N)
    nblk = N // blk
    blkl = min(512, N)
    nblkl = N // blkl
    centers = jnp.asarray(
        np.linspace(0.0, CUTOFF, R, dtype=np.float32).reshape(1, R))
    gamma = 1.0 / (float(CUTOFF) / (R - 1)) ** 2

    row2 = lambda a: a.reshape(1, -1)
    wspec = lambda arr: pl.BlockSpec(arr.shape, lambda b, i: (0,) * arr.ndim)

    win = p['Win']
    bin_ = row2(p['bin'])
    knn = pl.pallas_call(
        functools.partial(_knn_embed_kernel, blk=blk, n=N),
        grid=(B, nblk),
        in_specs=[
            pl.BlockSpec((1, blk, 6), lambda b, i: (b, i, 0)),
            pl.BlockSpec((1, N, 6), lambda b, i: (b, 0, 0)),
            wspec(win), wspec(bin_),
        ],
        out_specs=[
            pl.BlockSpec((1, blk, D), lambda b, i: (b, i, 0)),
            pl.BlockSpec((1, blk, K), lambda b, i: (b, i, 0)),
            pl.BlockSpec((1, blk, K), lambda b, i: (b, i, 0)),
        ],
        out_shape=[
            jax.ShapeDtypeStruct((B, N, D), jnp.float32),
            jax.ShapeDtypeStruct((B, N, K), jnp.int32),
            jax.ShapeDtypeStruct((B, N, K), jnp.float32),
        ],
    )
    h, idx, dmat = knn(x, x, win, bin_)
    sc_gather = _make_sc_gather(B * N * K, D)
    idxr = idx.reshape(B * N * K // 128, 128)

    for l in range(L):
        we1 = p['We1'][l]
        weights = [
            we1[:D], we1[D:2 * D], we1[2 * D:], row2(p['be1'][l]),
            p['We2'][l], row2(p['be2'][l]), p['Wu'][l], row2(p['bu'][l]),
            row2(p['lnL_g'][l]), row2(p['lnL_b'][l]),
            p['Wp'][l], row2(p['bp'][l]), p['Wm'][l], row2(p['bm'][l]),
            p['Wi'][l], row2(p['bi'][l]),
            row2(p['lnG_g'][l]), row2(p['lnG_b'][l]),
            p['Wf'][l][:D], p['Wf'][l][D:], row2(p['bf'][l]),
            p['Wc1'][l], row2(p['bc1'][l]), p['Wc2'][l], row2(p['bc2'][l]),
            p['Wg'][l], jnp.tile(p['bg'][l], D // 8).reshape(1, D),
            row2(p['n1_g'][l]), row2(p['n1_b'][l]),
            row2(p['n2_g'][l]), row2(p['n2_b'][l]),
        ]
        hj = sc_gather(h.reshape(B * N, D), idxr).reshape(B, N, K, D)
        layer = pl.pallas_call(
            functools.partial(_layer_kernel, blk=blkl,
                              gamma=gamma, d_model=D),
            grid=(B, nblkl),
            in_specs=[
                pl.BlockSpec((1, blkl, D), lambda b, i: (b, i, 0)),
                pl.BlockSpec((1, N, D), lambda b, i: (b, 0, 0)),
                pl.BlockSpec((1, blkl, K, D), lambda b, i: (b, i, 0, 0)),
                pl.BlockSpec((1, blkl, K), lambda b, i: (b, i, 0)),
                wspec(centers),
            ] + [wspec(w) for w in weights],
            out_specs=pl.BlockSpec((1, blkl, D), lambda b, i: (b, i, 0)),
            out_shape=jax.ShapeDtypeStruct((B, N, D), jnp.float32),
        )
        h = layer(h, h, hj, dmat, centers, *weights)

    wout = p['Wout']
    bout = row2(p['bout'])
    out = pl.pallas_call(
        _out_kernel,
        grid=(B, nblk),
        in_specs=[
            pl.BlockSpec((1, blk, 6), lambda b, i: (b, i, 0)),
            pl.BlockSpec((1, blk, D), lambda b, i: (b, i, 0)),
            wspec(wout), wspec(bout),
        ],
        out_specs=pl.BlockSpec((1, blk, 6), lambda b, i: (b, i, 0)),
        out_shape=jax.ShapeDtypeStruct((B, N, 6), jnp.float32),
    )
    return out(x, h, wout, bout)


# speedup vs baseline: 1.1948x; 1.1948x over previous
import functools
import jax
import jax.numpy as jnp
import numpy as np
from jax.experimental import pallas as pl
from jax.experimental.pallas import tpu as pltpu
from jax.experimental.pallas import tpu_sc as plsc

K = 16
CUTOFF = 5.0
SCALE = 0.1
LN_EPS = 1e-5


def _gelu(v):
    return 0.5 * v * (1.0 + jax.lax.erf(v * np.float32(0.7071067811865476)))


def _ln(v, g, b):
    mu = jnp.mean(v, axis=-1, keepdims=True)
    var = jnp.mean((v - mu) * (v - mu), axis=-1, keepdims=True)
    return (v - mu) * jax.lax.rsqrt(var + LN_EPS) * g + b


def _knn_embed_kernel(x_blk_ref, x_full_ref, win_ref, bin_ref,
                      h_ref, idx_ref, d_ref, *, blk, n):
    b = pl.program_id(0)
    i = pl.program_id(1)
    xb = x_blk_ref[0]
    xf = x_full_ref[0]
    pos_b = xb[:, :3]
    pos_f = xf[:, :3]
    sq_b = jnp.sum(pos_b * pos_b, axis=1)
    sq_f = jnp.sum(pos_f * pos_f, axis=1)
    cross = jax.lax.dot_general(pos_b, pos_f, (((1,), (1,)), ((), ())),
                                preferred_element_type=jnp.float32)
    d2 = sq_b[:, None] + sq_f[None, :] - 2.0 * cross
    cols = jax.lax.broadcasted_iota(jnp.int32, (blk, n), 1)
    rows = jax.lax.broadcasted_iota(jnp.int32, (blk, n), 0) + i * blk
    d2 = jnp.where(cols == rows, d2 + 1e9, d2)
    idxs = []
    vals = []
    for _ in range(K):
        m = jnp.min(d2, axis=1)
        cand = jnp.where(d2 <= m[:, None], cols, n)
        a = jnp.min(cand, axis=1)
        idxs.append(a[:, None])
        vals.append(m[:, None])
        d2 = jnp.where(cols == a[:, None], jnp.float32(3e38), d2)
    idx_ref[0] = jnp.concatenate(idxs, axis=1) + b * n
    d2k = jnp.concatenate(vals, axis=1)
    d_ref[0] = jnp.sqrt(jnp.maximum(d2k, 1e-12))
    h_ref[0] = jnp.dot(xb, win_ref[...], preferred_element_type=jnp.float32) + bin_ref[...]


def _layer_kernel(h_blk_ref, h_full_ref, hj_ref, d_ref, centers_ref,
                  we1a_ref, we1b_ref, we1r_ref, be1_ref,
                  we2_ref, be2_ref, wu_ref, bu_ref, lnlg_ref, lnlb_ref,
                  wp_ref, bp_ref, wm_ref, bm_ref, wi_ref, bi_ref,
                  lngg_ref, lngb_ref, wf1_ref, wf2_ref, bf_ref,
                  wc1_ref, bc1_ref, wc2_ref, bc2_ref,
                  wg_ref, bgt_ref, n1g_ref, n1b_ref, n2g_ref, n2b_ref,
                  out_ref, *, blk, gamma, d_model):
    D = d_model
    h = h_blk_ref[0]
    hj = hj_ref[0]
    d = d_ref[0]

    r_centers = centers_ref[...][0]
    rbf = jnp.exp(-gamma * (d[:, :, None] - r_centers) ** 2)
    nr = r_centers.shape[-1]
    hj2 = hj.reshape(blk * K, D)
    t = jax.lax.dot_general(hj2, we1b_ref[...], (((1,), (0,)), ((), ())),
                            preferred_element_type=jnp.float32)
    t = t + jax.lax.dot_general(rbf.reshape(blk * K, nr), we1r_ref[...],
                                (((1,), (0,)), ((), ())),
                                preferred_element_type=jnp.float32)
    hi_part = jnp.dot(h, we1a_ref[...], preferred_element_type=jnp.float32)
    pre = t.reshape(blk, K, D) + hi_part[:, None, :] + be1_ref[...][None]
    m1 = pre * jax.nn.sigmoid(pre)
    m = jnp.dot(m1.reshape(blk * K, D), we2_ref[...],
                preferred_element_type=jnp.float32) + be2_ref[...]
    mask = (d <= CUTOFF).astype(jnp.float32)
    m = m.reshape(blk, K, D) * mask[:, :, None]
    agg = jnp.sum(m, axis=1)
    local = _ln(h + jnp.dot(agg, wu_ref[...], preferred_element_type=jnp.float32)
                + bu_ref[...], lnlg_ref[...], lnlb_ref[...])

    hf = h_full_ref[0]
    mf = jnp.mean(hf, axis=0, keepdims=True)
    pmv = jnp.dot(h, wp_ref[...], preferred_element_type=jnp.float32) + bp_ref[...]
    mmv = jnp.dot(mf, wm_ref[...], preferred_element_type=jnp.float32) + bm_ref[...]
    mmv_t = jnp.concatenate([mmv] * 8, axis=1)
    rows8 = jax.lax.broadcasted_iota(jnp.int32, (8, 64), 0)
    cols64 = jax.lax.broadcasted_iota(jnp.int32, (8, 64), 1)
    msel = (cols64 // 8 == rows8).astype(jnp.float32)
    M = mmv_t * msel
    Q = jnp.dot(M, wi_ref[...], preferred_element_type=jnp.float32)
    glob_delta = jnp.dot(pmv, Q, preferred_element_type=jnp.float32) + bi_ref[...]
    glob = _ln(h + SCALE * glob_delta, lngg_ref[...], lngb_ref[...])

    fpre = (jnp.dot(local, wf1_ref[...], preferred_element_type=jnp.float32)
            + jnp.dot(glob, wf2_ref[...], preferred_element_type=jnp.float32)
            + bf_ref[...])
    fused = _gelu(fpre)

    v = _ln(fused, n1g_ref[...], n1b_ref[...])
    v = _gelu(jnp.dot(v, wc1_ref[...], preferred_element_type=jnp.float32)
              + bc1_ref[...])
    v = jnp.dot(v, wc2_ref[...], preferred_element_type=jnp.float32) + bc2_ref[...]
    v = v + fused

    ng = D // 8
    rid = jax.lax.broadcasted_iota(jnp.int32, (D, D), 0)
    cid = jax.lax.broadcasted_iota(jnp.int32, (D, D), 1)
    same_grp = (rid // 8 == cid // 8)
    geo = jnp.zeros((blk, D), jnp.float32)
    wg = wg_ref[...]
    for i in range(8):
        ei = (same_grp & (rid % 8 == i)).astype(jnp.float32)
        vi = jnp.dot(v, ei, preferred_element_type=jnp.float32)
        wgi = wg[8 * i:8 * (i + 1), :]
        tiled = jnp.concatenate([jnp.concatenate([wgi] * ng, axis=1)] * ng, axis=0)
        di = tiled * same_grp.astype(jnp.float32)
        ti = jnp.dot(v, di, preferred_element_type=jnp.float32)
        geo = geo + vi * ti
    geo = geo + bgt_ref[...]
    out_ref[0] = _ln(v + SCALE * geo, n2g_ref[...], n2b_ref[...])


def _make_sc_gather(total_rows, d_model):
    info = plsc.get_sparse_core_info()
    nc, ns = info.num_cores, info.num_subcores
    nw = nc * ns
    irows = total_rows // 128
    rpw = irows // nw
    mesh = plsc.VectorSubcoreMesh(core_axis_name="c", subcore_axis_name="s")

    assert rpw % 2 == 0

    @functools.partial(
        pl.kernel, mesh=mesh,
        out_type=jax.ShapeDtypeStruct((total_rows, d_model), jnp.float32),
        scratch_types=[
            pltpu.VMEM((rpw, 128), jnp.int32),
            pltpu.VMEM((128, d_model), jnp.float32),
            pltpu.VMEM((128, d_model), jnp.float32),
            pltpu.SemaphoreType.DMA,
            pltpu.SemaphoreType.DMA,
        ],
    )
    def gather(table_hbm, idx_hbm, out_hbm, idx_v, rows0, rows1, sem0, sem1):
        c = jax.lax.axis_index("c")
        s = jax.lax.axis_index("s")
        wid = s * nc + c
        pltpu.sync_copy(idx_hbm.at[pl.ds(wid * rpw, rpw)], idx_v)
        obase = wid * rpw * 128
        pltpu.async_copy(table_hbm.at[idx_v.at[0]], rows0, sem0)

        def body(t, carry):
            j0 = 2 * t
            pltpu.async_copy(table_hbm.at[idx_v.at[j0 + 1]], rows1, sem1)
            pltpu.make_async_copy(
                table_hbm.at[idx_v.at[j0]], rows0, sem0).wait()
            pltpu.sync_copy(rows0, out_hbm.at[pl.ds(obase + j0 * 128, 128)])

            @pl.when(t < rpw // 2 - 1)
            def _():
                pltpu.async_copy(table_hbm.at[idx_v.at[j0 + 2]], rows0, sem0)

            pltpu.make_async_copy(
                table_hbm.at[idx_v.at[j0 + 1]], rows1, sem1).wait()
            pltpu.sync_copy(
                rows1, out_hbm.at[pl.ds(obase + (j0 + 1) * 128, 128)])
            return carry

        jax.lax.fori_loop(0, rpw // 2, body, 0)

    return gather


def _out_kernel(x_blk_ref, h_blk_ref, wout_ref, bout_ref, o_ref):
    o_ref[0] = (x_blk_ref[0]
                + jnp.dot(h_blk_ref[0], wout_ref[...],
                          preferred_element_type=jnp.float32)
                + bout_ref[...])


def kernel(x, params):
    p = params
    B, N, _ = x.shape
    D = p['Win'].shape[1]
    L = p['We1'].shape[0]
    R = p['We1'].shape[1] - 2 * D
    blk = min(256, N)
    nblk = N // blk
    blkl = min(512, N)
    nblkl = N // blkl
    centers = jnp.asarray(
        np.linspace(0.0, CUTOFF, R, dtype=np.float32).reshape(1, R))
    gamma = 1.0 / (float(CUTOFF) / (R - 1)) ** 2

    row2 = lambda a: a.reshape(1, -1)
    wspec = lambda arr: pl.BlockSpec(arr.shape, lambda b, i: (0,) * arr.ndim)

    win = p['Win']
    bin_ = row2(p['bin'])
    knn = pl.pallas_call(
        functools.partial(_knn_embed_kernel, blk=blk, n=N),
        grid=(B, nblk),
        in_specs=[
            pl.BlockSpec((1, blk, 6), lambda b, i: (b, i, 0)),
            pl.BlockSpec((1, N, 6), lambda b, i: (b, 0, 0)),
            wspec(win), wspec(bin_),
        ],
        out_specs=[
            pl.BlockSpec((1, blk, D), lambda b, i: (b, i, 0)),
            pl.BlockSpec((1, blk, K), lambda b, i: (b, i, 0)),
            pl.BlockSpec((1, blk, K), lambda b, i: (b, i, 0)),
        ],
        out_shape=[
            jax.ShapeDtypeStruct((B, N, D), jnp.float32),
            jax.ShapeDtypeStruct((B, N, K), jnp.int32),
            jax.ShapeDtypeStruct((B, N, K), jnp.float32),
        ],
    )
    h, idx, dmat = knn(x, x, win, bin_)
    sc_gather = _make_sc_gather(B * N * K, D)
    idxr = idx.reshape(B * N * K // 128, 128)

    for l in range(L):
        we1 = p['We1'][l]
        weights = [
            we1[:D], we1[D:2 * D], we1[2 * D:], row2(p['be1'][l]),
            p['We2'][l], row2(p['be2'][l]), p['Wu'][l], row2(p['bu'][l]),
            row2(p['lnL_g'][l]), row2(p['lnL_b'][l]),
            p['Wp'][l], row2(p['bp'][l]), p['Wm'][l], row2(p['bm'][l]),
            p['Wi'][l], row2(p['bi'][l]),
            row2(p['lnG_g'][l]), row2(p['lnG_b'][l]),
            p['Wf'][l][:D], p['Wf'][l][D:], row2(p['bf'][l]),
            p['Wc1'][l], row2(p['bc1'][l]), p['Wc2'][l], row2(p['bc2'][l]),
            p['Wg'][l], jnp.tile(p['bg'][l], D // 8).reshape(1, D),
            row2(p['n1_g'][l]), row2(p['n1_b'][l]),
            row2(p['n2_g'][l]), row2(p['n2_b'][l]),
        ]
        hj = sc_gather(h.reshape(B * N, D), idxr).reshape(B, N, K, D)
        layer = pl.pallas_call(
            functools.partial(_layer_kernel, blk=blkl,
                              gamma=gamma, d_model=D),
            grid=(B, nblkl),
            in_specs=[
                pl.BlockSpec((1, blkl, D), lambda b, i: (b, i, 0)),
                pl.BlockSpec((1, N, D), lambda b, i: (b, 0, 0)),
                pl.BlockSpec((1, blkl, K, D), lambda b, i: (b, i, 0, 0)),
                pl.BlockSpec((1, blkl, K), lambda b, i: (b, i, 0)),
                wspec(centers),
            ] + [wspec(w) for w in weights],
            out_specs=pl.BlockSpec((1, blkl, D), lambda b, i: (b, i, 0)),
            out_shape=jax.ShapeDtypeStruct((B, N, D), jnp.float32),
        )
        h = layer(h, h, hj, dmat, centers, *weights)

    wout = p['Wout']
    bout = row2(p['bout'])
    out = pl.pallas_call(
        _out_kernel,
        grid=(B, nblk),
        in_specs=[
            pl.BlockSpec((1, blk, 6), lambda b, i: (b, i, 0)),
            pl.BlockSpec((1, blk, D), lambda b, i: (b, i, 0)),
            wspec(wout), wspec(bout),
        ],
        out_specs=pl.BlockSpec((1, blk, 6), lambda b, i: (b, i, 0)),
        out_shape=jax.ShapeDtypeStruct((B, N, 6), jnp.float32),
    )
    return out(x, h, wout, bout)


# speedup vs baseline: 1.2654x; 1.0591x over previous
import functools
import jax
import jax.numpy as jnp
import numpy as np
from jax.experimental import pallas as pl
from jax.experimental.pallas import tpu as pltpu
from jax.experimental.pallas import tpu_sc as plsc

K = 16
CUTOFF = 5.0
SCALE = 0.1
LN_EPS = 1e-5


def _gelu(v):
    return 0.5 * v * (1.0 + jax.lax.erf(v * np.float32(0.7071067811865476)))


def _ln(v, g, b):
    mu = jnp.mean(v, axis=-1, keepdims=True)
    var = jnp.mean((v - mu) * (v - mu), axis=-1, keepdims=True)
    return (v - mu) * jax.lax.rsqrt(var + LN_EPS) * g + b


def _knn_embed_kernel(x_blk_ref, x_full_ref, win_ref, bin_ref,
                      h_ref, idx_ref, d_ref, *, blk, n):
    b = pl.program_id(0)
    i = pl.program_id(1)
    xb = x_blk_ref[0]
    xf = x_full_ref[0]
    pos_b = xb[:, :3]
    pos_f = xf[:, :3]
    sq_b = jnp.sum(pos_b * pos_b, axis=1)
    sq_f = jnp.sum(pos_f * pos_f, axis=1)
    cross = jax.lax.dot_general(pos_b, pos_f, (((1,), (1,)), ((), ())),
                                preferred_element_type=jnp.float32)
    d2 = sq_b[:, None] + sq_f[None, :] - 2.0 * cross
    cols = jax.lax.broadcasted_iota(jnp.int32, (blk, n), 1)
    rows = jax.lax.broadcasted_iota(jnp.int32, (blk, n), 0) + i * blk
    d2 = jnp.where(cols == rows, d2 + 1e9, d2)
    idxs = []
    vals = []
    for _ in range(K):
        m = jnp.min(d2, axis=1)
        cand = jnp.where(d2 <= m[:, None], cols, n)
        a = jnp.min(cand, axis=1)
        idxs.append(a[:, None])
        vals.append(m[:, None])
        d2 = jnp.where(cols == a[:, None], jnp.float32(3e38), d2)
    idx_ref[0] = jnp.concatenate(idxs, axis=1) + b * n
    d2k = jnp.concatenate(vals, axis=1)
    d_ref[0] = jnp.sqrt(jnp.maximum(d2k, 1e-12))
    h_ref[0] = jnp.dot(xb, win_ref[...], preferred_element_type=jnp.float32) + bin_ref[...]


def _layer_kernel(h_blk_ref, h_full_ref, hj_ref, d_ref, centers_ref,
                  we1a_ref, we1b_ref, we1r_ref, be1_ref,
                  we2_ref, be2_ref, wu_ref, bu_ref, lnlg_ref, lnlb_ref,
                  wp_ref, bp_ref, wm_ref, bm_ref, wi_ref, bi_ref,
                  lngg_ref, lngb_ref, wf1_ref, wf2_ref, bf_ref,
                  wc1_ref, bc1_ref, wc2_ref, bc2_ref,
                  wg_ref, bgt_ref, n1g_ref, n1b_ref, n2g_ref, n2b_ref,
                  out_ref, *, blk, gamma, d_model):
    D = d_model
    h = h_blk_ref[0]
    hj = hj_ref[0]
    d = d_ref[0]

    r_centers = centers_ref[...][0]
    rbf = jnp.exp(-gamma * (d[:, :, None] - r_centers) ** 2)
    nr = r_centers.shape[-1]
    hj2 = hj.reshape(blk * K, D)
    t = jax.lax.dot_general(hj2, we1b_ref[...], (((1,), (0,)), ((), ())),
                            preferred_element_type=jnp.float32)
    t = t + jax.lax.dot_general(rbf.reshape(blk * K, nr), we1r_ref[...],
                                (((1,), (0,)), ((), ())),
                                preferred_element_type=jnp.float32)
    hi_part = jnp.dot(h, we1a_ref[...], preferred_element_type=jnp.float32)
    pre = t.reshape(blk, K, D) + hi_part[:, None, :] + be1_ref[...][None]
    m1 = pre * jax.nn.sigmoid(pre)
    m = jnp.dot(m1.reshape(blk * K, D), we2_ref[...],
                preferred_element_type=jnp.float32) + be2_ref[...]
    mask = (d <= CUTOFF).astype(jnp.float32)
    m = m.reshape(blk, K, D) * mask[:, :, None]
    agg = jnp.sum(m, axis=1)
    local = _ln(h + jnp.dot(agg, wu_ref[...], preferred_element_type=jnp.float32)
                + bu_ref[...], lnlg_ref[...], lnlb_ref[...])

    hf = h_full_ref[0]
    mf = jnp.mean(hf, axis=0, keepdims=True)
    pmv = jnp.dot(h, wp_ref[...], preferred_element_type=jnp.float32) + bp_ref[...]
    mmv = jnp.dot(mf, wm_ref[...], preferred_element_type=jnp.float32) + bm_ref[...]
    mmv_t = jnp.concatenate([mmv] * 8, axis=1)
    rows8 = jax.lax.broadcasted_iota(jnp.int32, (8, 64), 0)
    cols64 = jax.lax.broadcasted_iota(jnp.int32, (8, 64), 1)
    msel = (cols64 // 8 == rows8).astype(jnp.float32)
    M = mmv_t * msel
    Q = jnp.dot(M, wi_ref[...], preferred_element_type=jnp.float32)
    glob_delta = jnp.dot(pmv, Q, preferred_element_type=jnp.float32) + bi_ref[...]
    glob = _ln(h + SCALE * glob_delta, lngg_ref[...], lngb_ref[...])

    fpre = (jnp.dot(local, wf1_ref[...], preferred_element_type=jnp.float32)
            + jnp.dot(glob, wf2_ref[...], preferred_element_type=jnp.float32)
            + bf_ref[...])
    fused = _gelu(fpre)

    v = _ln(fused, n1g_ref[...], n1b_ref[...])
    v = _gelu(jnp.dot(v, wc1_ref[...], preferred_element_type=jnp.float32)
              + bc1_ref[...])
    v = jnp.dot(v, wc2_ref[...], preferred_element_type=jnp.float32) + bc2_ref[...]
    v = v + fused

    ng = D // 8
    rid = jax.lax.broadcasted_iota(jnp.int32, (D, D), 0)
    cid = jax.lax.broadcasted_iota(jnp.int32, (D, D), 1)
    same_grp = (rid // 8 == cid // 8)
    geo = jnp.zeros((blk, D), jnp.float32)
    wg = wg_ref[...]
    for i in range(8):
        ei = (same_grp & (rid % 8 == i)).astype(jnp.float32)
        vi = jnp.dot(v, ei, preferred_element_type=jnp.float32)
        wgi = wg[8 * i:8 * (i + 1), :]
        tiled = jnp.concatenate([jnp.concatenate([wgi] * ng, axis=1)] * ng, axis=0)
        di = tiled * same_grp.astype(jnp.float32)
        ti = jnp.dot(v, di, preferred_element_type=jnp.float32)
        geo = geo + vi * ti
    geo = geo + bgt_ref[...]
    out_ref[0] = _ln(v + SCALE * geo, n2g_ref[...], n2b_ref[...])


def _make_sc_gather(total_rows, d_model):
    info = plsc.get_sparse_core_info()
    nc, ns = info.num_cores, info.num_subcores
    nw = nc * ns
    irows = total_rows // 128
    rpw = irows // nw
    mesh = plsc.VectorSubcoreMesh(core_axis_name="c", subcore_axis_name="s")

    assert rpw % 2 == 0

    @functools.partial(
        pl.kernel, mesh=mesh,
        out_type=jax.ShapeDtypeStruct((total_rows, d_model), jnp.float32),
        scratch_types=[
            pltpu.VMEM((rpw, 128), jnp.int32),
            pltpu.VMEM((128, d_model), jnp.float32),
            pltpu.VMEM((128, d_model), jnp.float32),
            pltpu.SemaphoreType.DMA,
            pltpu.SemaphoreType.DMA,
        ],
    )
    def gather(table_hbm, idx_hbm, out_hbm, idx_v, rows0, rows1, sem0, sem1):
        c = jax.lax.axis_index("c")
        s = jax.lax.axis_index("s")
        wid = s * nc + c
        pltpu.sync_copy(idx_hbm.at[pl.ds(wid * rpw, rpw)], idx_v)
        obase = wid * rpw * 128
        pltpu.async_copy(table_hbm.at[idx_v.at[0]], rows0, sem0)

        def body(t, carry):
            j0 = 2 * t
            pltpu.async_copy(table_hbm.at[idx_v.at[j0 + 1]], rows1, sem1)
            pltpu.make_async_copy(
                table_hbm.at[idx_v.at[j0]], rows0, sem0).wait()
            pltpu.sync_copy(rows0, out_hbm.at[pl.ds(obase + j0 * 128, 128)])

            @pl.when(t < rpw // 2 - 1)
            def _():
                pltpu.async_copy(table_hbm.at[idx_v.at[j0 + 2]], rows0, sem0)

            pltpu.make_async_copy(
                table_hbm.at[idx_v.at[j0 + 1]], rows1, sem1).wait()
            pltpu.sync_copy(
                rows1, out_hbm.at[pl.ds(obase + (j0 + 1) * 128, 128)])
            return carry

        jax.lax.fori_loop(0, rpw // 2, body, 0)

    return gather


def _out_kernel(x_blk_ref, h_blk_ref, wout_ref, bout_ref, o_ref):
    o_ref[0] = (x_blk_ref[0]
                + jnp.dot(h_blk_ref[0], wout_ref[...],
                          preferred_element_type=jnp.float32)
                + bout_ref[...])


def kernel(x, params):
    p = params
    B, N, _ = x.shape
    D = p['Win'].shape[1]
    L = p['We1'].shape[0]
    R = p['We1'].shape[1] - 2 * D
    blk = min(256, N)
    nblk = N // blk
    blkl = min(1024, N)
    nblkl = N // blkl
    centers = jnp.asarray(
        np.linspace(0.0, CUTOFF, R, dtype=np.float32).reshape(1, R))
    gamma = 1.0 / (float(CUTOFF) / (R - 1)) ** 2

    row2 = lambda a: a.reshape(1, -1)
    wspec = lambda arr: pl.BlockSpec(arr.shape, lambda b, i: (0,) * arr.ndim)

    win = p['Win']
    bin_ = row2(p['bin'])
    knn = pl.pallas_call(
        functools.partial(_knn_embed_kernel, blk=blk, n=N),
        grid=(B, nblk),
        in_specs=[
            pl.BlockSpec((1, blk, 6), lambda b, i: (b, i, 0)),
            pl.BlockSpec((1, N, 6), lambda b, i: (b, 0, 0)),
            wspec(win), wspec(bin_),
        ],
        out_specs=[
            pl.BlockSpec((1, blk, D), lambda b, i: (b, i, 0)),
            pl.BlockSpec((1, blk, K), lambda b, i: (b, i, 0)),
            pl.BlockSpec((1, blk, K), lambda b, i: (b, i, 0)),
        ],
        out_shape=[
            jax.ShapeDtypeStruct((B, N, D), jnp.float32),
            jax.ShapeDtypeStruct((B, N, K), jnp.int32),
            jax.ShapeDtypeStruct((B, N, K), jnp.float32),
        ],
    )
    h, idx, dmat = knn(x, x, win, bin_)
    sc_gather = _make_sc_gather(B * N * K, D)
    idxr = idx.reshape(B * N * K // 128, 128)

    for l in range(L):
        we1 = p['We1'][l]
        weights = [
            we1[:D], we1[D:2 * D], we1[2 * D:], row2(p['be1'][l]),
            p['We2'][l], row2(p['be2'][l]), p['Wu'][l], row2(p['bu'][l]),
            row2(p['lnL_g'][l]), row2(p['lnL_b'][l]),
            p['Wp'][l], row2(p['bp'][l]), p['Wm'][l], row2(p['bm'][l]),
            p['Wi'][l], row2(p['bi'][l]),
            row2(p['lnG_g'][l]), row2(p['lnG_b'][l]),
            p['Wf'][l][:D], p['Wf'][l][D:], row2(p['bf'][l]),
            p['Wc1'][l], row2(p['bc1'][l]), p['Wc2'][l], row2(p['bc2'][l]),
            p['Wg'][l], jnp.tile(p['bg'][l], D // 8).reshape(1, D),
            row2(p['n1_g'][l]), row2(p['n1_b'][l]),
            row2(p['n2_g'][l]), row2(p['n2_b'][l]),
        ]
        hj = sc_gather(h.reshape(B * N, D), idxr).reshape(B, N, K, D)
        layer = pl.pallas_call(
            functools.partial(_layer_kernel, blk=blkl,
                              gamma=gamma, d_model=D),
            grid=(B, nblkl),
            in_specs=[
                pl.BlockSpec((1, blkl, D), lambda b, i: (b, i, 0)),
                pl.BlockSpec((1, N, D), lambda b, i: (b, 0, 0)),
                pl.BlockSpec((1, blkl, K, D), lambda b, i: (b, i, 0, 0)),
                pl.BlockSpec((1, blkl, K), lambda b, i: (b, i, 0)),
                wspec(centers),
            ] + [wspec(w) for w in weights],
            out_specs=pl.BlockSpec((1, blkl, D), lambda b, i: (b, i, 0)),
            out_shape=jax.ShapeDtypeStruct((B, N, D), jnp.float32),
        )
        h = layer(h, h, hj, dmat, centers, *weights)

    wout = p['Wout']
    bout = row2(p['bout'])
    out = pl.pallas_call(
        _out_kernel,
        grid=(B, nblk),
        in_specs=[
            pl.BlockSpec((1, blk, 6), lambda b, i: (b, i, 0)),
            pl.BlockSpec((1, blk, D), lambda b, i: (b, i, 0)),
            wspec(wout), wspec(bout),
        ],
        out_specs=pl.BlockSpec((1, blk, 6), lambda b, i: (b, i, 0)),
        out_shape=jax.ShapeDtypeStruct((B, N, 6), jnp.float32),
    )
    return out(x, h, wout, bout)
